# baseline (speedup 1.0000x reference)
import functools

import jax
import jax.numpy as jnp
from jax import lax
from jax.experimental import pallas as pl
from jax.experimental.pallas import tpu as pltpu

_M, _B, _C = 4, 4096, 2048
_BT = 256
_RT = 16


def _rotl(x, r):
    return lax.shift_left(x, jnp.int32(r)) + lax.shift_right_logical(
        x, jnp.int32(32 - r))


_ROT0 = (13, 15, 26, 6)
_ROT1 = (17, 29, 16, 24)


def _threefry_bits(lo):
    k1 = jnp.int32(42)
    k2 = jnp.int32(0x1BD11BDA) ^ k1
    ks = (jnp.int32(0), k1, k2)
    x0 = jnp.zeros_like(lo)
    x1 = lo + k1
    rots = (_ROT0, _ROT1, _ROT0, _ROT1, _ROT0)
    kidx = ((1, 2), (2, 0), (0, 1), (1, 2), (2, 0))
    for g in range(5):
        for r in rots[g]:
            x0 = x0 + x1
            x1 = _rotl(x1, r)
            x1 = x1 ^ x0
        a, b = kidx[g]
        x0 = x0 + ks[a]
        x1 = x1 + ks[b] + jnp.int32(g + 1)
    return x0 ^ x1


def _embrace_kernel(x_ref, o_ref):
    b0 = pl.program_id(0) * _BT

    def chunk(c_i, _):
        r0 = c_i * _RT
        row = lax.broadcasted_iota(jnp.int32, (_RT, _C), 0)
        col = lax.broadcasted_iota(jnp.int32, (_RT, _C), 1)
        base = ((b0 + r0 + row) * _C + col) * _M

        planes = []
        best = None
        idx = None
        for m in range(_M):
            xm = x_ref[m, pl.ds(r0, _RT), :]
            planes.append(xm)
            hd = jnp.any(xm != 0.0, axis=1, keepdims=True)
            shifted = lax.shift_right_logical(
                _threefry_bits(base + jnp.int32(m)), jnp.int32(9))
            key = jnp.where(hd, shifted + 1, 0)
            if m == 0:
                best = key
                idx = jnp.zeros((_RT, _C), jnp.int32)
            else:
                gt = key > best
                idx = jnp.where(gt, jnp.int32(m), idx)
                best = jnp.maximum(key, best)

        e = planes[3]
        for m in (2, 1, 0):
            e = jnp.where(idx == m, planes[m], e)
        o_ref[pl.ds(r0, _RT), :] = e
        return _

    lax.fori_loop(0, _BT // _RT, chunk, 0, unroll=False)


@jax.jit
def kernel(x):
    grid = _B // _BT
    return pl.pallas_call(
        _embrace_kernel,
        grid=(grid,),
        in_specs=[pl.BlockSpec((_M, _BT, _C), lambda i: (0, i, 0))],
        out_specs=pl.BlockSpec((_BT, _C), lambda i: (i, 0)),
        out_shape=jax.ShapeDtypeStruct((_B, _C), x.dtype),
    )(x)


# speedup vs baseline: 1.1183x; 1.1183x over previous
import functools

import jax
import jax.numpy as jnp
from jax import lax
from jax.experimental import pallas as pl
from jax.experimental.pallas import tpu as pltpu

_M, _B, _C = 4, 4096, 2048
_BT = 256
_RT = 8


def _rotl(x, r):
    return lax.shift_left(x, jnp.int32(r)) + lax.shift_right_logical(
        x, jnp.int32(32 - r))


_ROT0 = (13, 15, 26, 6)
_ROT1 = (17, 29, 16, 24)


def _threefry_bits(lo):
    k1 = jnp.int32(42)
    k2 = jnp.int32(0x1BD11BDA) ^ k1
    ks = (jnp.int32(0), k1, k2)
    x0 = jnp.zeros_like(lo)
    x1 = lo + k1
    rots = (_ROT0, _ROT1, _ROT0, _ROT1, _ROT0)
    kidx = ((1, 2), (2, 0), (0, 1), (1, 2), (2, 0))
    for g in range(5):
        for r in rots[g]:
            x0 = x0 + x1
            x1 = _rotl(x1, r)
            x1 = x1 ^ x0
        a, b = kidx[g]
        x0 = x0 + ks[a]
        x1 = x1 + ks[b] + jnp.int32(g + 1)
    return x0 ^ x1


def _embrace_kernel(x_ref, o_ref):
    b0 = pl.program_id(0) * _BT

    def chunk(c_i, _):
        r0 = c_i * _RT
        row = lax.broadcasted_iota(jnp.int32, (_RT, _C), 0)
        col = lax.broadcasted_iota(jnp.int32, (_RT, _C), 1)
        base = ((b0 + r0 + row) * _C + col) * _M

        planes = []
        best = None
        idx = None
        for m in range(_M):
            xm = x_ref[m, pl.ds(r0, _RT), :]
            planes.append(xm)
            hd = jnp.any(xm != 0.0, axis=1, keepdims=True)
            shifted = lax.shift_right_logical(
                _threefry_bits(base + jnp.int32(m)), jnp.int32(9))
            key = jnp.where(hd, shifted + 1, 0)
            if m == 0:
                best = key
                idx = jnp.zeros((_RT, _C), jnp.int32)
            else:
                gt = key > best
                idx = jnp.where(gt, jnp.int32(m), idx)
                best = jnp.maximum(key, best)

        e = planes[3]
        for m in (2, 1, 0):
            e = jnp.where(idx == m, planes[m], e)
        o_ref[pl.ds(r0, _RT), :] = e
        return _

    lax.fori_loop(0, _BT // _RT, chunk, 0, unroll=2)


@jax.jit
def kernel(x):
    grid = _B // _BT
    return pl.pallas_call(
        _embrace_kernel,
        grid=(grid,),
        in_specs=[pl.BlockSpec((_M, _BT, _C), lambda i: (0, i, 0))],
        out_specs=pl.BlockSpec((_BT, _C), lambda i: (i, 0)),
        out_shape=jax.ShapeDtypeStruct((_B, _C), x.dtype),
    )(x)


# speedup vs baseline: 1.1277x; 1.0084x over previous
import functools

import jax
import jax.numpy as jnp
from jax import lax
from jax.experimental import pallas as pl
from jax.experimental.pallas import tpu as pltpu

_M, _B, _C = 4, 4096, 2048
_BT = 512
_RT = 8


def _rotl(x, r):
    return lax.shift_left(x, jnp.int32(r)) + lax.shift_right_logical(
        x, jnp.int32(32 - r))


_ROT0 = (13, 15, 26, 6)
_ROT1 = (17, 29, 16, 24)


def _threefry_bits(lo):
    k1 = jnp.int32(42)
    k2 = jnp.int32(0x1BD11BDA) ^ k1
    ks = (jnp.int32(0), k1, k2)
    x0 = jnp.zeros_like(lo)
    x1 = lo + k1
    rots = (_ROT0, _ROT1, _ROT0, _ROT1, _ROT0)
    kidx = ((1, 2), (2, 0), (0, 1), (1, 2), (2, 0))
    for g in range(5):
        for r in rots[g]:
            x0 = x0 + x1
            x1 = _rotl(x1, r)
            x1 = x1 ^ x0
        a, b = kidx[g]
        x0 = x0 + ks[a]
        x1 = x1 + ks[b] + jnp.int32(g + 1)
    return x0 ^ x1


def _embrace_kernel(x_ref, o_ref):
    b0 = pl.program_id(0) * _BT

    def chunk(c_i, _):
        r0 = c_i * _RT
        row = lax.broadcasted_iota(jnp.int32, (_RT, _C), 0)
        col = lax.broadcasted_iota(jnp.int32, (_RT, _C), 1)
        base = ((b0 + r0 + row) * _C + col) * _M

        best = None
        e = None
        for m in range(_M):
            xm = x_ref[m, pl.ds(r0, _RT), :]
            hd = jnp.any(xm != 0.0, axis=1, keepdims=True)
            shifted = lax.shift_right_logical(
                _threefry_bits(base + jnp.int32(m)), jnp.int32(9))
            key = jnp.where(hd, shifted + 1, 0)
            if m == 0:
                best = key
                e = xm
            else:
                gt = key > best
                e = jnp.where(gt, xm, e)
                best = jnp.maximum(key, best)

        o_ref[pl.ds(r0, _RT), :] = e
        return _

    lax.fori_loop(0, _BT // _RT, chunk, 0, unroll=2)


@jax.jit
def kernel(x):
    grid = _B // _BT
    return pl.pallas_call(
        _embrace_kernel,
        grid=(grid,),
        in_specs=[pl.BlockSpec((_M, _BT, _C), lambda i: (0, i, 0))],
        out_specs=pl.BlockSpec((_BT, _C), lambda i: (i, 0)),
        out_shape=jax.ShapeDtypeStruct((_B, _C), x.dtype),
    )(x)


# speedup vs baseline: 1.1286x; 1.0008x over previous
import jax
import jax.numpy as jnp
from jax import lax
from jax.experimental import pallas as pl

_M, _B, _C = 4, 4096, 2048
_RT = 8


def _rotl(x, r):
    return lax.shift_left(x, jnp.int32(r)) + lax.shift_right_logical(
        x, jnp.int32(32 - r))


_ROT0 = (13, 15, 26, 6)
_ROT1 = (17, 29, 16, 24)


def _threefry_bits(lo):
    k1 = jnp.int32(42)
    k2 = jnp.int32(0x1BD11BDA) ^ k1
    ks = (jnp.int32(0), k1, k2)
    x0 = jnp.zeros_like(lo)
    x1 = lo + k1
    rots = (_ROT0, _ROT1, _ROT0, _ROT1, _ROT0)
    kidx = ((1, 2), (2, 0), (0, 1), (1, 2), (2, 0))
    for g in range(5):
        for r in rots[g]:
            x0 = x0 + x1
            x1 = _rotl(x1, r)
            x1 = x1 ^ x0
        a, b = kidx[g]
        x0 = x0 + ks[a]
        x1 = x1 + ks[b] + jnp.int32(g + 1)
    return x0 ^ x1


def _embrace_kernel(x_ref, o_ref):
    b0 = pl.program_id(0) * _RT
    row = lax.broadcasted_iota(jnp.int32, (_RT, _C), 0)
    col = lax.broadcasted_iota(jnp.int32, (_RT, _C), 1)
    base = ((b0 + row) * _C + col) * _M

    best = None
    e = None
    for m in range(_M):
        xm = x_ref[m]
        hd = jnp.any(xm != 0.0, axis=1, keepdims=True)
        shifted = lax.shift_right_logical(
            _threefry_bits(base + jnp.int32(m)), jnp.int32(9))
        key = jnp.where(hd, shifted + 1, 0)
        if m == 0:
            best = key
            e = xm
        else:
            gt = key > best
            e = jnp.where(gt, xm, e)
            best = jnp.maximum(key, best)
    o_ref[...] = e


@jax.jit
def kernel(x):
    grid = _B // _RT
    return pl.pallas_call(
        _embrace_kernel,
        grid=(grid,),
        in_specs=[pl.BlockSpec((_M, _RT, _C), lambda i: (0, i, 0))],
        out_specs=pl.BlockSpec((_RT, _C), lambda i: (i, 0)),
        out_shape=jax.ShapeDtypeStruct((_B, _C), x.dtype),
    )(x)


# speedup vs baseline: 1.1450x; 1.0146x over previous
import jax
import jax.numpy as jnp
from jax import lax
from jax.experimental import pallas as pl

_M, _B, _C = 4, 4096, 2048
_RT = 8


def _rotl(x, r):
    return lax.shift_left(x, jnp.int32(r)) + lax.shift_right_logical(
        x, jnp.int32(32 - r))


_ROT0 = (13, 15, 26, 6)
_ROT1 = (17, 29, 16, 24)


def _threefry_bits(lo):
    k1 = jnp.int32(42)
    k2 = jnp.int32(0x1BD11BDA) ^ k1
    ks = (jnp.int32(0), k1, k2)
    x1 = lo + k1
    x0 = x1
    x1 = _rotl(x1, 13) ^ x0
    rots = ((15, 26, 6),) + (_ROT1, _ROT0, _ROT1, _ROT0)
    kidx = ((1, 2), (2, 0), (0, 1), (1, 2), (2, 0))
    for g in range(5):
        for r in rots[g]:
            x0 = x0 + x1
            x1 = _rotl(x1, r)
            x1 = x1 ^ x0
        a, b = kidx[g]
        if a != 0:
            x0 = x0 + ks[a]
        x1 = x1 + (ks[b] + jnp.int32(g + 1))
    return x0 ^ x1


def _embrace_kernel(x_ref, o_ref):
    b0 = pl.program_id(0) * _RT
    row = lax.broadcasted_iota(jnp.int32, (_RT, _C), 0)
    col = lax.broadcasted_iota(jnp.int32, (_RT, _C), 1)
    base = ((b0 + row) * _C + col) * _M

    best = None
    e = None
    for m in range(_M):
        xm = x_ref[m]
        hd = jnp.any(xm != 0.0, axis=1, keepdims=True)
        shifted = lax.shift_right_logical(
            _threefry_bits(base + jnp.int32(m)), jnp.int32(9))
        key = jnp.where(hd, shifted + 1, 0)
        if m == 0:
            best = key
            e = xm
        else:
            gt = key > best
            e = jnp.where(gt, xm, e)
            best = jnp.maximum(key, best)
    o_ref[...] = e


@jax.jit
def kernel(x):
    grid = _B // _RT
    return pl.pallas_call(
        _embrace_kernel,
        grid=(grid,),
        in_specs=[pl.BlockSpec((_M, _RT, _C), lambda i: (0, i, 0))],
        out_specs=pl.BlockSpec((_RT, _C), lambda i: (i, 0)),
        out_shape=jax.ShapeDtypeStruct((_B, _C), x.dtype),
    )(x)


# speedup vs baseline: 1.1602x; 1.0133x over previous
import jax
import jax.numpy as jnp
from jax import lax
from jax.experimental import pallas as pl

_M, _B, _C = 4, 4096, 2048
_RT = 8


def _rotl(x, r):
    return lax.shift_left(x, jnp.int32(r)) + lax.shift_right_logical(
        x, jnp.int32(32 - r))


_ROT0 = (13, 15, 26, 6)
_ROT1 = (17, 29, 16, 24)


def _threefry_bits(lo):
    k1 = jnp.int32(42)
    k2 = jnp.int32(0x1BD11BDA) ^ k1
    ks = (jnp.int32(0), k1, k2)
    x1 = lo + k1
    x0 = x1
    x1 = _rotl(x1, 13) ^ x0
    rots = ((15, 26, 6),) + (_ROT1, _ROT0, _ROT1, _ROT0)
    kidx = ((1, 2), (2, 0), (0, 1), (1, 2), (2, 0))
    for g in range(5):
        for r in rots[g]:
            x0 = x0 + x1
            x1 = _rotl(x1, r)
            x1 = x1 ^ x0
        a, b = kidx[g]
        if a != 0:
            x0 = x0 + ks[a]
        x1 = x1 + (ks[b] + jnp.int32(g + 1))
    return x0 ^ x1


def _embrace_kernel(x_ref, o_ref):
    b0 = pl.program_id(0) * _RT
    row = lax.broadcasted_iota(jnp.int32, (_RT, _C), 0)
    col = lax.broadcasted_iota(jnp.int32, (_RT, _C), 1)
    base = ((b0 + row) * _C + col) * _M

    best = None
    e = None
    for m in range(_M):
        xm = x_ref[m]
        hd = jnp.any(xm != 0.0, axis=1, keepdims=True)
        shifted = lax.shift_right_logical(
            _threefry_bits(base + jnp.int32(m)), jnp.int32(9))
        key = jnp.where(hd, shifted, -1)
        if m == 0:
            best = key
            e = xm
        else:
            gt = key > best
            e = jnp.where(gt, xm, e)
            best = jnp.maximum(key, best)
    o_ref[...] = e


@jax.jit
def kernel(x):
    grid = _B // _RT
    return pl.pallas_call(
        _embrace_kernel,
        grid=(grid,),
        in_specs=[pl.BlockSpec((_M, _RT, _C), lambda i: (0, i, 0))],
        out_specs=pl.BlockSpec((_RT, _C), lambda i: (i, 0)),
        out_shape=jax.ShapeDtypeStruct((_B, _C), x.dtype),
    )(x)


# speedup vs baseline: 1.1607x; 1.0004x over previous
import jax
import jax.numpy as jnp
from jax import lax
from jax.experimental import pallas as pl
from jax.experimental.pallas import tpu as pltpu

_M, _B, _C = 4, 4096, 2048
_RT = 8


def _rotl(x, r):
    return lax.shift_left(x, jnp.int32(r)) + lax.shift_right_logical(
        x, jnp.int32(32 - r))


_ROT0 = (13, 15, 26, 6)
_ROT1 = (17, 29, 16, 24)


def _threefry_bits(lo):
    k1 = jnp.int32(42)
    k2 = jnp.int32(0x1BD11BDA) ^ k1
    ks = (jnp.int32(0), k1, k2)
    x1 = lo + k1
    x0 = x1
    x1 = _rotl(x1, 13) ^ x0
    rots = ((15, 26, 6),) + (_ROT1, _ROT0, _ROT1, _ROT0)
    kidx = ((1, 2), (2, 0), (0, 1), (1, 2), (2, 0))
    for g in range(5):
        for r in rots[g]:
            x0 = x0 + x1
            x1 = _rotl(x1, r)
            x1 = x1 ^ x0
        a, b = kidx[g]
        if a != 0:
            x0 = x0 + ks[a]
        x1 = x1 + (ks[b] + jnp.int32(g + 1))
    return x0 ^ x1


def _embrace_kernel(x_ref, o_ref):
    b0 = pl.program_id(0) * _RT
    row = lax.broadcasted_iota(jnp.int32, (_RT, _C), 0)
    col = lax.broadcasted_iota(jnp.int32, (_RT, _C), 1)
    base = ((b0 + row) * _C + col) * _M

    best = None
    e = None
    for m in range(_M):
        xm = x_ref[m]
        hd = jnp.any(xm != 0.0, axis=1, keepdims=True)
        shifted = lax.shift_right_logical(
            _threefry_bits(base + jnp.int32(m)), jnp.int32(9))
        key = jnp.where(hd, shifted, -1)
        if m == 0:
            best = key
            e = xm
        else:
            gt = key > best
            e = jnp.where(gt, xm, e)
            best = jnp.maximum(key, best)
    o_ref[...] = e


@jax.jit
def kernel(x):
    grid = _B // _RT
    return pl.pallas_call(
        _embrace_kernel,
        grid=(grid,),
        in_specs=[pl.BlockSpec((_M, _RT, _C), lambda i: (0, i, 0))],
        out_specs=pl.BlockSpec((_RT, _C), lambda i: (i, 0)),
        out_shape=jax.ShapeDtypeStruct((_B, _C), x.dtype),
        compiler_params=pltpu.CompilerParams(
            dimension_semantics=("parallel",),
        ),
    )(x)


# speedup vs baseline: 1.1679x; 1.0062x over previous
import jax
import jax.numpy as jnp
from jax import lax
from jax.experimental import pallas as pl
from jax.experimental.pallas import tpu as pltpu

_M, _B, _C = 4, 4096, 2048
_RT = 8


def _rotl(x, r):
    return lax.shift_left(x, jnp.int32(r)) + lax.shift_right_logical(
        x, jnp.int32(32 - r))


_ROT0 = (13, 15, 26, 6)
_ROT1 = (17, 29, 16, 24)


def _threefry_bits(lo):
    k1 = jnp.int32(42)
    k2 = jnp.int32(0x1BD11BDA) ^ k1
    ks = (jnp.int32(0), k1, k2)
    x1 = lo + k1
    x0 = x1
    x1 = _rotl(x1, 13) ^ x0
    rots = ((15, 26, 6),) + (_ROT1, _ROT0, _ROT1, _ROT0)
    kidx = ((1, 2), (2, 0), (0, 1), (1, 2), (2, 0))
    for g in range(5):
        for r in rots[g]:
            x0 = x0 + x1
            x1 = _rotl(x1, r)
            x1 = x1 ^ x0
        a, b = kidx[g]
        if a != 0:
            x0 = x0 + ks[a]
        x1 = x1 + (ks[b] + jnp.int32(g + 1))
    return x0 ^ x1


def _embrace_kernel(x_ref, o_ref):
    b0 = pl.program_id(0) * (2 * _RT)
    row = lax.broadcasted_iota(jnp.int32, (_RT, _C), 0)
    col = lax.broadcasted_iota(jnp.int32, (_RT, _C), 1)

    for half in range(2):
        r0 = half * _RT
        base = ((b0 + r0 + row) * _C + col) * _M
        best = None
        e = None
        for m in range(_M):
            xm = x_ref[m, pl.ds(r0, _RT), :]
            hd = jnp.any(xm != 0.0, axis=1, keepdims=True)
            shifted = lax.shift_right_logical(
                _threefry_bits(base + jnp.int32(m)), jnp.int32(9))
            key = jnp.where(hd, shifted, -1)
            if m == 0:
                best = key
                e = xm
            else:
                gt = key > best
                e = jnp.where(gt, xm, e)
                best = jnp.maximum(key, best)
        o_ref[pl.ds(r0, _RT), :] = e


@jax.jit
def kernel(x):
    grid = _B // (2 * _RT)
    return pl.pallas_call(
        _embrace_kernel,
        grid=(grid,),
        in_specs=[pl.BlockSpec((_M, 2 * _RT, _C), lambda i: (0, i, 0))],
        out_specs=pl.BlockSpec((2 * _RT, _C), lambda i: (i, 0)),
        out_shape=jax.ShapeDtypeStruct((_B, _C), x.dtype),
        compiler_params=pltpu.CompilerParams(
            dimension_semantics=("parallel",),
        ),
    )(x)


# speedup vs baseline: 1.1732x; 1.0045x over previous
import jax
import jax.numpy as jnp
from jax import lax
from jax.experimental import pallas as pl
from jax.experimental.pallas import tpu as pltpu

_M, _B, _C = 4, 4096, 2048
_RT = 8


def _rotl(x, r):
    return lax.shift_left(x, jnp.int32(r)) + lax.shift_right_logical(
        x, jnp.int32(32 - r))


_ROT0 = (13, 15, 26, 6)
_ROT1 = (17, 29, 16, 24)


def _threefry_bits(lo):
    k1 = jnp.int32(42)
    k2 = jnp.int32(0x1BD11BDA) ^ k1
    ks = (jnp.int32(0), k1, k2)
    x1 = lo + k1
    x0 = x1
    x1 = _rotl(x1, 13) ^ x0
    rots = ((15, 26, 6),) + (_ROT1, _ROT0, _ROT1, _ROT0)
    kidx = ((1, 2), (2, 0), (0, 1), (1, 2), (2, 0))
    for g in range(5):
        for r in rots[g]:
            x0 = x0 + x1
            x1 = _rotl(x1, r)
            x1 = x1 ^ x0
        a, b = kidx[g]
        if a != 0:
            x0 = x0 + ks[a]
        x1 = x1 + (ks[b] + jnp.int32(g + 1))
    return x0 ^ x1


def _embrace_kernel(x_ref, o_ref):
    b0 = pl.program_id(0) * (4 * _RT)
    row = lax.broadcasted_iota(jnp.int32, (_RT, _C), 0)
    col = lax.broadcasted_iota(jnp.int32, (_RT, _C), 1)

    for half in range(4):
        r0 = half * _RT
        base = ((b0 + r0 + row) * _C + col) * _M
        best = None
        e = None
        for m in range(_M):
            xm = x_ref[m, pl.ds(r0, _RT), :]
            hd = jnp.any(xm != 0.0, axis=1, keepdims=True)
            shifted = lax.shift_right_logical(
                _threefry_bits(base + jnp.int32(m)), jnp.int32(9))
            key = jnp.where(hd, shifted, -1)
            if m == 0:
                best = key
                e = xm
            else:
                gt = key > best
                e = jnp.where(gt, xm, e)
                best = jnp.maximum(key, best)
        o_ref[pl.ds(r0, _RT), :] = e


@jax.jit
def kernel(x):
    grid = _B // (4 * _RT)
    return pl.pallas_call(
        _embrace_kernel,
        grid=(grid,),
        in_specs=[pl.BlockSpec((_M, 4 * _RT, _C), lambda i: (0, i, 0))],
        out_specs=pl.BlockSpec((4 * _RT, _C), lambda i: (i, 0)),
        out_shape=jax.ShapeDtypeStruct((_B, _C), x.dtype),
        compiler_params=pltpu.CompilerParams(
            dimension_semantics=("parallel",),
        ),
    )(x)
